# baseline (speedup 1.0000x reference)
import functools

import jax
import jax.numpy as jnp
import numpy as np
from jax.experimental import pallas as pl
from jax.experimental.pallas import tpu as pltpu

N = 10000
K = 16
G = 16

NEG_INF = float("-inf")


def _in_mlp_body(x_ref, w1_ref, b1_ref, w2_ref, b2_ref, w3_ref, b3_ref, o_ref):
    h = x_ref[...]
    h = jnp.maximum(jnp.dot(h, w1_ref[...], preferred_element_type=jnp.float32)
                    + b1_ref[...], 0.0)
    h = jnp.maximum(jnp.dot(h, w2_ref[...], preferred_element_type=jnp.float32)
                    + b2_ref[...], 0.0)
    h = jnp.maximum(jnp.dot(h, w3_ref[...], preferred_element_type=jnp.float32)
                    + b3_ref[...], 0.0)
    o_ref[...] = h


def _in_mlp(x, w1, b1, w2, b2, w3, b3):
    BLK = 2000
    grid = (N // BLK,)
    return pl.pallas_call(
        _in_mlp_body,
        grid=grid,
        in_specs=[
            pl.BlockSpec((BLK, 5), lambda i: (i, 0)),
            pl.BlockSpec((5, 128), lambda i: (0, 0)),
            pl.BlockSpec((128,), lambda i: (0,)),
            pl.BlockSpec((128, 128), lambda i: (0, 0)),
            pl.BlockSpec((128,), lambda i: (0,)),
            pl.BlockSpec((128, 64), lambda i: (0, 0)),
            pl.BlockSpec((64,), lambda i: (0,)),
        ],
        out_specs=pl.BlockSpec((BLK, 64), lambda i: (i, 0)),
        out_shape=jax.ShapeDtypeStruct((N, 64), jnp.float32),
    )(x, w1, b1, w2, b2, w3, b3)


_KNN_BLK = 200


def _knn_body(xr_ref, xct_ref, br_ref, bc_ref, idx_ref, val_ref, sim_ref):
    i = pl.program_id(0)
    B = xr_ref.shape[0]
    col = jax.lax.broadcasted_iota(jnp.int32, (B, N), 1)
    row = jax.lax.broadcasted_iota(jnp.int32, (B, N), 0) + i * B
    same = br_ref[...] == bc_ref[...]
    sim = jnp.dot(xr_ref[...], xct_ref[...], preferred_element_type=jnp.float32)
    sim_ref[...] = jnp.where(same & (col != row), sim, NEG_INF)
    for k in range(K):
        s = sim_ref[...]
        m = jnp.max(s, axis=1)
        isel = jnp.min(jnp.where(s == m[:, None], col, N), axis=1)
        idx_ref[:, k] = isel
        val_ref[:, k] = m
        sim_ref[...] = jnp.where(col == isel[:, None], NEG_INF, s)


def _knn_topk(xn, batch_row, batch_col):
    F = xn.shape[1]
    B = _KNN_BLK
    grid = (N // B,)
    return pl.pallas_call(
        _knn_body,
        grid=grid,
        in_specs=[
            pl.BlockSpec((B, F), lambda i: (i, 0)),
            pl.BlockSpec((F, N), lambda i: (0, 0)),
            pl.BlockSpec((B, 1), lambda i: (i, 0)),
            pl.BlockSpec((1, N), lambda i: (0, 0)),
        ],
        out_specs=[
            pl.BlockSpec((B, K), lambda i: (i, 0)),
            pl.BlockSpec((B, K), lambda i: (i, 0)),
        ],
        out_shape=[
            jax.ShapeDtypeStruct((N, K), jnp.int32),
            jax.ShapeDtypeStruct((N, K), jnp.float32),
        ],
        scratch_shapes=[pltpu.VMEM((B, N), jnp.float32)],
    )(xn, xn.T, batch_row, batch_col)


def _knn_undirected_p(x, batch_row, batch_col):
    xn = x / (jnp.linalg.norm(x, axis=1, keepdims=True) + 1e-12)
    nbr2, val2 = _knn_topk(xn, batch_row, batch_col)
    ok2 = val2 > -1e30
    valid_nbr = jnp.where(ok2, nbr2, -1)
    mut = jnp.any(valid_nbr[nbr2] == jnp.arange(N)[:, None, None], axis=-1)
    nbr = nbr2.reshape(-1)
    ok = ok2.reshape(-1)
    mutf = mut.reshape(-1)
    ctr = jnp.repeat(jnp.arange(N), K)
    SENT = N * N
    e1 = jnp.where(ok & ~mutf, nbr * N + ctr, SENT)
    e2 = jnp.where(ok, ctr * N + nbr, SENT)
    eids = jnp.sort(jnp.concatenate([e1, e2]))
    ev = eids < SENT
    src = jnp.where(ev, eids // N, 0)
    dst = jnp.where(ev, eids % N, 0)
    return src, dst, ev


def _edge_conv_x(x, src, dst, ev, w1, b1, g, be, w2, b2):
    xi = x[dst]
    xj = x[src]
    h = jnp.concatenate([xi, xj - xi], axis=-1)
    h = h @ w1 + b1
    h = h / jnp.sqrt(1.0 + 1e-5) * g + be
    h = jax.nn.relu(h)
    h = jax.nn.relu(h @ w2 + b2)
    h = h * ev[:, None].astype(h.dtype)
    return jax.ops.segment_sum(h, dst, num_segments=N)


def kernel(x, batch, in_w1, in_b1, in_w2, in_b2, in_w3, in_b3, c1_w1, c1_b1, c1_g, c1_be, c1_w2, c1_b2, c2_w1, c2_b1, c2_g, c2_be, c2_w2, c2_b2, c3_w1, c3_b1, c3_g, c3_be, c3_w2, c3_b2, out_w1, out_b1, out_w2, out_b2, out_w3, out_b3):
    batch_row = batch.reshape(N, 1).astype(jnp.int32)
    batch_col = batch.reshape(1, N).astype(jnp.int32)
    h = _in_mlp(x, in_w1, in_b1, in_w2, in_b2, in_w3, in_b3)
    orig = h
    src, dst, ev = _knn_undirected_p(h, batch_row, batch_col)
    h = _edge_conv_x(h, src, dst, ev, c1_w1, c1_b1, c1_g, c1_be, c1_w2, c1_b2)
    h = jnp.concatenate([h, orig], axis=-1)
    res1 = h
    src, dst, ev = _knn_undirected_p(h, batch_row, batch_col)
    h = _edge_conv_x(h, src, dst, ev, c2_w1, c2_b1, c2_g, c2_be, c2_w2, c2_b2)
    h = jnp.concatenate([h, res1], axis=-1)
    res2 = h
    src, dst, ev = _knn_undirected_p(h, batch_row, batch_col)
    h = _edge_conv_x(h, src, dst, ev, c3_w1, c3_b1, c3_g, c3_be, c3_w2, c3_b2)
    h = jnp.concatenate([h, res2], axis=-1)
    pooled = jax.ops.segment_max(h, batch, num_segments=G)
    pooled = jnp.where(jnp.isfinite(pooled), pooled, 0.0)
    o = jax.nn.relu(pooled @ out_w1 + out_b1)
    o = jax.nn.relu(o @ out_w2 + out_b2)
    o = o @ out_w3 + out_b3
    return (o.squeeze(-1), h, jnp.stack([src, dst]))
